# baseline (device time: 189860 ns/iter reference)
import jax
import jax.numpy as jnp
from jax import lax
from jax.experimental import pallas as pl
from jax.experimental.pallas import tpu as pltpu

N_DEV = 16
N_STREAMS = 4

RING = (0, 1, 5, 9, 13, 14, 10, 6, 2, 3, 7, 11, 15, 12, 8, 4)


def kernel(x, w_mat, scale_x, scale_w):
    m_total, k = x.shape
    _, n = w_mat.shape
    m_per = m_total // N_DEV
    half = n // 2
    sub = n // N_STREAMS

    ring = jnp.array(RING, dtype=jnp.int32)
    my = lax.axis_index("i")
    p = jnp.argmax(ring == my).astype(jnp.int32)
    nxt = ring[(p + 1) % N_DEV]
    prv = ring[(p - 1) % N_DEV]
    nbrs = jnp.stack([nxt, prv])
    s_idx = jnp.arange(N_DEV, dtype=jnp.int32)
    cf_all = ring[(p - 1 - s_idx) % N_DEV]
    cb_all = ring[(p + 1 + s_idx) % N_DEV]

    def body(x_ref, w_ref, sx_ref, sw_ref, nbr_ref, cf_ref, cb_ref,
             out_ref, comm_ref, send_sems, recv_sems):
        nxt_ = nbr_ref[0]
        prv_ = nbr_ref[1]

        barrier_sem = pltpu.get_barrier_semaphore()
        for nbr in (nxt_, prv_):
            pl.semaphore_signal(
                barrier_sem, inc=1,
                device_id=(nbr,), device_id_type=pl.DeviceIdType.MESH,
            )
        pl.semaphore_wait(barrier_sem, 2)

        def xs(c):
            return x_ref[pl.ds(c * m_per, m_per), :]

        def pfwd(c):
            return jnp.dot(xs(c), w_ref[:, 0:half],
                           preferred_element_type=jnp.float32)

        def pbwd(c):
            return jnp.dot(xs(c), w_ref[:, half:n],
                           preferred_element_type=jnp.float32)

        dests = (nxt_, nxt_, prv_, prv_)

        def mk(k_, s_):
            return pltpu.make_async_remote_copy(
                src_ref=comm_ref.at[k_, s_],
                dst_ref=comm_ref.at[k_, s_ + 1],
                send_sem=send_sems.at[k_, s_],
                recv_sem=recv_sems.at[k_, s_ + 1],
                device_id=(dests[k_],),
                device_id_type=pl.DeviceIdType.MESH,
            )

        pf0 = pfwd(cf_ref[0])
        pb0 = pbwd(cb_ref[0])
        comm_ref[0, 0, :, :] = pf0[:, 0:sub]
        comm_ref[1, 0, :, :] = pf0[:, sub:half]
        comm_ref[2, 0, :, :] = pb0[:, 0:sub]
        comm_ref[3, 0, :, :] = pb0[:, sub:half]

        for k_ in range(N_STREAMS):
            mk(k_, 0).start()

        scale = sx_ref[0] * sw_ref[0]

        for s in range(N_DEV - 1):
            pf = pfwd(cf_ref[s + 1])
            pb = pbwd(cb_ref[s + 1])
            parts = (pf[:, 0:sub], pf[:, sub:half],
                     pb[:, 0:sub], pb[:, sub:half])
            for k_ in (0, 2, 1, 3):
                h = mk(k_, s)
                h.wait()
                if s < N_DEV - 2:
                    comm_ref[k_, s + 1, :, :] = (
                        comm_ref[k_, s + 1, :, :] + parts[k_]
                    )
                    mk(k_, s + 1).start()
                else:
                    col0 = k_ * sub
                    out_ref[:, col0:col0 + sub] = (
                        comm_ref[k_, s + 1, :, :] + parts[k_]
                    ) * scale

    return pl.pallas_call(
        body,
        out_shape=jax.ShapeDtypeStruct((m_per, n), jnp.float32),
        in_specs=[
            pl.BlockSpec(memory_space=pltpu.VMEM),
            pl.BlockSpec(memory_space=pltpu.VMEM),
            pl.BlockSpec(memory_space=pltpu.SMEM),
            pl.BlockSpec(memory_space=pltpu.SMEM),
            pl.BlockSpec(memory_space=pltpu.SMEM),
            pl.BlockSpec(memory_space=pltpu.SMEM),
            pl.BlockSpec(memory_space=pltpu.SMEM),
        ],
        out_specs=pl.BlockSpec(memory_space=pltpu.VMEM),
        scratch_shapes=[
            pltpu.VMEM((N_STREAMS, N_DEV, m_per, sub), jnp.float32),
            pltpu.SemaphoreType.DMA((N_STREAMS, N_DEV)),
            pltpu.SemaphoreType.DMA((N_STREAMS, N_DEV)),
        ],
        compiler_params=pltpu.CompilerParams(
            collective_id=0, vmem_limit_bytes=100 * 1024 * 1024
        ),
    )(x, w_mat, scale_x, scale_w, nbrs, cf_all, cb_all)


# device time: 100853 ns/iter; 1.8825x vs baseline; 1.8825x over previous
import jax
import jax.numpy as jnp
from jax import lax
from jax.experimental import pallas as pl
from jax.experimental.pallas import tpu as pltpu

N_DEV = 16
N_STREAMS = 4


def kernel(x, w_mat, scale_x, scale_w):
    m_total, k = x.shape
    _, n = w_mat.shape
    m_per = m_total // N_DEV
    half = n // 2
    sub = n // N_STREAMS

    def body(x_ref, w_ref, sx_ref, sw_ref, out_ref, comm_ref, send_sems, recv_sems):
        my = lax.axis_index("i")
        left = lax.rem(my + N_DEV - 1, N_DEV)
        right = lax.rem(my + 1, N_DEV)

        barrier_sem = pltpu.get_barrier_semaphore()
        for nbr in (left, right):
            pl.semaphore_signal(
                barrier_sem, inc=1,
                device_id=(nbr,), device_id_type=pl.DeviceIdType.MESH,
            )
        pl.semaphore_wait(barrier_sem, 2)

        def xs(c):
            return x_ref[pl.ds(c * m_per, m_per), :]

        def pfwd(c):
            return jnp.dot(xs(c), w_ref[:, 0:half],
                           preferred_element_type=jnp.float32)

        def pbwd(c):
            return jnp.dot(xs(c), w_ref[:, half:n],
                           preferred_element_type=jnp.float32)

        dests = (right, right, left, left)

        def mk(k_, s_):
            return pltpu.make_async_remote_copy(
                src_ref=comm_ref.at[k_, s_],
                dst_ref=comm_ref.at[k_, s_ + 1],
                send_sem=send_sems.at[k_, s_],
                recv_sem=recv_sems.at[k_, s_ + 1],
                device_id=(dests[k_],),
                device_id_type=pl.DeviceIdType.MESH,
            )

        pf0 = pfwd(left).astype(jnp.bfloat16)
        pb0 = pbwd(right).astype(jnp.bfloat16)
        comm_ref[0, 0, :, :] = pf0[:, 0:sub]
        comm_ref[1, 0, :, :] = pf0[:, sub:half]
        comm_ref[2, 0, :, :] = pb0[:, 0:sub]
        comm_ref[3, 0, :, :] = pb0[:, sub:half]

        for k_ in range(N_STREAMS):
            mk(k_, 0).start()

        scale = sx_ref[0] * sw_ref[0]

        for s in range(N_DEV - 1):
            cf = lax.rem(my + 2 * N_DEV - 2 - s, N_DEV)
            cb = lax.rem(my + 2 + s, N_DEV)
            pf = pfwd(cf)
            pb = pbwd(cb)
            parts = (pf[:, 0:sub], pf[:, sub:half],
                     pb[:, 0:sub], pb[:, sub:half])
            for k_ in (0, 2, 1, 3):
                h = mk(k_, s)
                h.wait()
                acc = comm_ref[k_, s + 1, :, :].astype(jnp.float32) + parts[k_]
                if s < N_DEV - 2:
                    comm_ref[k_, s + 1, :, :] = acc.astype(jnp.bfloat16)
                    mk(k_, s + 1).start()
                else:
                    col0 = k_ * sub
                    out_ref[:, col0:col0 + sub] = acc * scale

    return pl.pallas_call(
        body,
        out_shape=jax.ShapeDtypeStruct((m_per, n), jnp.float32),
        in_specs=[
            pl.BlockSpec(memory_space=pltpu.VMEM),
            pl.BlockSpec(memory_space=pltpu.VMEM),
            pl.BlockSpec(memory_space=pltpu.SMEM),
            pl.BlockSpec(memory_space=pltpu.SMEM),
        ],
        out_specs=pl.BlockSpec(memory_space=pltpu.VMEM),
        scratch_shapes=[
            pltpu.VMEM((N_STREAMS, N_DEV, m_per, sub), jnp.bfloat16),
            pltpu.SemaphoreType.DMA((N_STREAMS, N_DEV)),
            pltpu.SemaphoreType.DMA((N_STREAMS, N_DEV)),
        ],
        compiler_params=pltpu.CompilerParams(
            collective_id=0, vmem_limit_bytes=100 * 1024 * 1024
        ),
    )(x, w_mat, scale_x, scale_w)
